# baseline (speedup 1.0000x reference)
import functools

import jax
import jax.numpy as jnp
from jax import lax
from jax.experimental import pallas as pl
from jax.experimental.pallas import tpu as pltpu
from jax.experimental.pallas import tpu_sc as plsc

N = 10000
E = 320000
H = 128
NC = 2
NS = 16
NW = NC * NS
E_PER_W = E // NW
CH = 40
NCHUNK = E_PER_W // CH
ROWS_PER_SUB = 624
ROWS_TAIL = N - NS * ROWS_PER_SUB


NSLOT = 3
ISLOT = 6


def _edge_body(a_hbm, b_hbm, ea_hbm, w3_hbm, src_hbm, dst_hbm, zero_hbm,
               out_hbm, si, di, av, bv, ev0, ev1, ev2, w3v, agg_sh, *sems):
    evs = (ev0, ev1, ev2)
    c = lax.axis_index("c")
    s = lax.axis_index("s")
    wid = c * NS + s
    sem_i = sems[:ISLOT]
    sem_g = sems[ISLOT:ISLOT + NSLOT]
    sem_s = sems[ISLOT + NSLOT:]

    pltpu.sync_copy(w3_hbm, w3v)

    r0 = s * ROWS_PER_SUB
    pltpu.sync_copy(zero_hbm.at[pl.ds(r0, ROWS_PER_SUB)],
                    agg_sh.at[pl.ds(r0, ROWS_PER_SUB)])

    @pl.when(s == NS - 1)
    def _zero_tail():
        pltpu.sync_copy(zero_hbm.at[pl.ds(NS * ROWS_PER_SUB, ROWS_TAIL)],
                        agg_sh.at[pl.ds(NS * ROWS_PER_SUB, ROWS_TAIL)])

    plsc.subcore_barrier()

    def idx_cps(g, q):
        base = wid * E_PER_W + g * CH
        return (
            pltpu.make_async_copy(src_hbm.at[pl.ds(base, CH)], si.at[q], sem_i[q]),
            pltpu.make_async_copy(dst_hbm.at[pl.ds(base, CH)], di.at[q], sem_i[q]),
        )

    def gather_cps(g, k, q):
        base = wid * E_PER_W + g * CH
        return (
            pltpu.make_async_copy(a_hbm.at[si.at[q]], av.at[k], sem_g[k]),
            pltpu.make_async_copy(b_hbm.at[di.at[q]], bv.at[k], sem_g[k]),
            pltpu.make_async_copy(ea_hbm.at[pl.ds(base * 4, CH * 4 + 16)],
                                  evs[k], sem_g[k]),
        )

    def scatter_start(k, q):
        pltpu.async_copy(av.at[k], agg_sh.at[di.at[q]], sem_s[k], add=True)

    def scatter_wait(k, q):
        pltpu.make_async_copy(av.at[k], agg_sh.at[di.at[q]], sem_s[k]).wait()

    w3 = [[w3v[kk, pl.ds(j * 16, 16)] for j in range(H // 16)]
          for kk in range(4)]
    lane0 = [jnp.full((16, 1), kk, jnp.int32) for kk in range(4)]
    _dnums = lax.GatherDimensionNumbers(
        offset_dims=(), collapsed_slice_dims=(0,), start_index_map=(0,))

    def _splat(vec, idx):
        return lax.gather(vec, idx, _dnums, (1,),
                          mode=lax.GatherScatterMode.PROMISE_IN_BOUNDS)

    for g in range(3):
        for cp in idx_cps(g, g):
            cp.start()
    for g in range(2):
        for cp in idx_cps(g, g):
            cp.wait()
        for cp in gather_cps(g, g, g):
            cp.start()

    def six(i, carry):
        for u in range(ISLOT):
            g = i * ISLOT + u
            k = u % NSLOT
            q = u
            kp = (u + 2) % NSLOT
            qn = (u + 2) % ISLOT
            qf = (u + 3) % ISLOT

            @pl.when(g < NCHUNK)
            def _chunk():
                for cp in gather_cps(g, k, q):
                    cp.wait()

                @plsc.parallel_loop(0, CH, unroll=4)
                def _edge(e):
                    v4 = evs[k][pl.ds(e * 4, 16)]
                    spl = [_splat(v4, lane0[kk]) for kk in range(4)]
                    for j in range(H // 16):
                        sl = pl.ds(j * 16, 16)
                        v = av[k, e, sl] + bv[k, e, sl]
                        for kk in range(4):
                            v = v + spl[kk] * w3[kk][j]
                        av[k, e, sl] = jnp.maximum(v, 0.0)
                scatter_start(k, q)

                @pl.when(g >= 1)
                def _drain_prev():
                    scatter_wait(kp, (u + ISLOT - 1) % ISLOT)

                @pl.when(g + 3 < NCHUNK)
                def _idx_ahead():
                    for cp in idx_cps(g + 3, qf):
                        cp.start()

                @pl.when(g + 2 < NCHUNK)
                def _refill():
                    for cp in idx_cps(g + 2, qn):
                        cp.wait()
                    for cp in gather_cps(g + 2, kp, qn):
                        cp.start()

        return carry

    lax.fori_loop(0, (NCHUNK + ISLOT - 1) // ISLOT, six, 0)
    scatter_wait((NCHUNK - 1) % NSLOT, (NCHUNK - 1) % ISLOT)
    plsc.subcore_barrier()
    pltpu.sync_copy(agg_sh.at[pl.ds(r0, ROWS_PER_SUB)],
                    out_hbm.at[pl.ds(c * N + r0, ROWS_PER_SUB)])

    @pl.when(s == NS - 1)
    def _out_tail():
        pltpu.sync_copy(agg_sh.at[pl.ds(NS * ROWS_PER_SUB, ROWS_TAIL)],
                        out_hbm.at[pl.ds(c * N + NS * ROWS_PER_SUB, ROWS_TAIL)])


_edge_pass = functools.partial(
    pl.kernel,
    mesh=plsc.VectorSubcoreMesh(core_axis_name="c", subcore_axis_name="s"),
    out_type=jax.ShapeDtypeStruct((2 * N, H), jnp.float32),
    scratch_types=[
        pltpu.VMEM((ISLOT, CH), jnp.int32),
        pltpu.VMEM((ISLOT, CH), jnp.int32),
        pltpu.VMEM((NSLOT, CH, H), jnp.float32),
        pltpu.VMEM((NSLOT, CH, H), jnp.float32),
        pltpu.VMEM((CH * 4 + 16,), jnp.float32),
        pltpu.VMEM((CH * 4 + 16,), jnp.float32),
        pltpu.VMEM((CH * 4 + 16,), jnp.float32),
        pltpu.VMEM((8, H), jnp.float32),
        pltpu.VMEM_SHARED((N, H), jnp.float32),
    ] + [pltpu.SemaphoreType.DMA] * (ISLOT + 2 * NSLOT),
)(_edge_body)


_RB = 400
_GRID = N // _RB


def _enc_body(x, W_enc, b_enc, We1, We2, b_edge, h_o, a_o, b_o):
    h = jnp.dot(x[...], W_enc[...], preferred_element_type=jnp.float32) + b_enc[...]
    h_o[...] = h
    a_o[...] = jnp.dot(h, We1[...], preferred_element_type=jnp.float32) + b_edge[...]
    b_o[...] = jnp.dot(h, We2[...], preferred_element_type=jnp.float32)


def _enc_ab(x, W_enc, b_enc, We1, We2, b_edge):
    w = pl.BlockSpec((H, H), lambda i: (0, 0))
    b = pl.BlockSpec((1, H), lambda i: (0, 0))
    return pl.pallas_call(
        _enc_body,
        grid=(_GRID,),
        in_specs=[pl.BlockSpec((_RB, H), lambda i: (i, 0)), w, b, w, w, b],
        out_specs=[pl.BlockSpec((_RB, H), lambda i: (i, 0))] * 3,
        out_shape=[jax.ShapeDtypeStruct((N, H), jnp.float32)] * 3,
    )(x, W_enc, b_enc, We1, We2, b_edge)


def _node_body(h, p0, p1, Wn1, Wn2, b_node, We1, We2, b_edge, h_o, a_o, b_o):
    agg = p0[...] + p1[...]
    t = (jnp.dot(h[...], Wn1[...], preferred_element_type=jnp.float32)
         + jnp.dot(agg, Wn2[...], preferred_element_type=jnp.float32)
         + b_node[...])
    t = jnp.maximum(t, 0.0)
    h_o[...] = t
    a_o[...] = jnp.dot(t, We1[...], preferred_element_type=jnp.float32) + b_edge[...]
    b_o[...] = jnp.dot(t, We2[...], preferred_element_type=jnp.float32)


def _node_update(h, p0, p1, Wn1, Wn2, b_node, We1, We2, b_edge):
    r = pl.BlockSpec((_RB, H), lambda i: (i, 0))
    w = pl.BlockSpec((H, H), lambda i: (0, 0))
    b = pl.BlockSpec((1, H), lambda i: (0, 0))
    return pl.pallas_call(
        _node_body,
        grid=(_GRID,),
        in_specs=[r, r, r, w, w, b, w, w, b],
        out_specs=[r] * 3,
        out_shape=[jax.ShapeDtypeStruct((N, H), jnp.float32)] * 3,
    )(h, p0, p1, Wn1, Wn2, b_node, We1, We2, b_edge)


def _final_body(h, p0, p1, Wn1, Wn2, b_node, Wd1, bd1, Wd2, bd2, out_o):
    agg = p0[...] + p1[...]
    t = (jnp.dot(h[...], Wn1[...], preferred_element_type=jnp.float32)
         + jnp.dot(agg, Wn2[...], preferred_element_type=jnp.float32)
         + b_node[...])
    t = jnp.maximum(t, 0.0)
    z = jnp.maximum(
        jnp.dot(t, Wd1[...], preferred_element_type=jnp.float32) + bd1[...], 0.0)
    out_o[...] = jnp.dot(z, Wd2[...], preferred_element_type=jnp.float32) + bd2[...]


def _final_dec(h, p0, p1, Wn1, Wn2, b_node, Wd1, bd1, Wd2, bd2, DT, DO):
    r = pl.BlockSpec((_RB, H), lambda i: (i, 0))
    w = pl.BlockSpec((H, H), lambda i: (0, 0))
    return pl.pallas_call(
        _final_body,
        grid=(_GRID,),
        in_specs=[r, r, r, w, w, pl.BlockSpec((1, H), lambda i: (0, 0)),
                  pl.BlockSpec((H, DT), lambda i: (0, 0)),
                  pl.BlockSpec((1, DT), lambda i: (0, 0)),
                  pl.BlockSpec((DT, DO), lambda i: (0, 0)),
                  pl.BlockSpec((1, DO), lambda i: (0, 0))],
        out_specs=pl.BlockSpec((_RB, DO), lambda i: (i, 0)),
        out_shape=jax.ShapeDtypeStruct((N, DO), jnp.float32),
    )(h, p0, p1, Wn1, Wn2, b_node, Wd1, bd1, Wd2, bd2)


def kernel(x, edge_index, edge_attr, W_enc, b_enc, W_edge, b_edge,
           W_node, b_node, dec):
    We1 = W_edge[:H]
    We2 = W_edge[H:2 * H]
    We3 = jnp.concatenate([W_edge[2 * H:], jnp.zeros((4, H), jnp.float32)])
    Wn1 = W_node[:H]
    Wn2 = W_node[H:]
    src = edge_index[0]
    dst = edge_index[1]
    b_enc2 = b_enc.reshape(1, H)
    b_edge2 = b_edge.reshape(1, H)
    b_node2 = b_node.reshape(1, H)

    names = sorted(dec.keys())
    dims = [dec[n][2].shape[1] for n in names]
    DT = 64 * len(names)
    DO = sum(dims)
    Wd1 = jnp.concatenate([dec[n][0] for n in names], axis=1)
    bd1 = jnp.concatenate([dec[n][1] for n in names]).reshape(1, DT)
    Wd2 = jnp.zeros((DT, DO), jnp.float32)
    off = 0
    for i, n in enumerate(names):
        Wd2 = Wd2.at[64 * i:64 * (i + 1), off:off + dims[i]].set(dec[n][2])
        off += dims[i]
    bd2 = jnp.concatenate([dec[n][3] for n in names]).reshape(1, DO)

    zero = jnp.zeros((N, H), jnp.float32)
    ea_flat = jnp.concatenate([edge_attr.reshape(E * 4),
                               jnp.zeros((16,), jnp.float32)])

    h, A, B = _enc_ab(x, W_enc, b_enc2, We1, We2, b_edge2)

    for layer in range(3):
        parts = _edge_pass(A, B, ea_flat, We3, src, dst, zero)
        p0, p1 = parts[:N], parts[N:]
        if layer < 2:
            h, A, B = _node_update(h, p0, p1, Wn1, Wn2, b_node2, We1, We2,
                                   b_edge2)
        else:
            out = _final_dec(h, p0, p1, Wn1, Wn2, b_node2, Wd1, bd1, Wd2, bd2,
                             DT, DO)
    return out


# speedup vs baseline: 1.7746x; 1.7746x over previous
import functools

import jax
import jax.numpy as jnp
from jax import lax
from jax.experimental import pallas as pl
from jax.experimental.pallas import tpu as pltpu
from jax.experimental.pallas import tpu_sc as plsc

N = 10000
E = 320000
H = 128
NC = 2
NS = 16
NW = NC * NS
E_PER_W = E // NW
CH = 40
NCHUNK = E_PER_W // CH
ROWS_PER_SUB = 624
ROWS_TAIL = N - NS * ROWS_PER_SUB


NSLOT = 3
ISLOT = 6


def _edge_body(a_hbm, b_hbm, ea_hbm, w3_hbm, src_hbm, dst_hbm, zero_hbm,
               out_hbm, si, di, av, bv, ev0, ev1, ev2, w3v, agg_sh, *sems):
    evs = (ev0, ev1, ev2)
    c = lax.axis_index("c")
    s = lax.axis_index("s")
    wid = c * NS + s
    sem_i = sems[:ISLOT]
    sem_g = sems[ISLOT:ISLOT + NSLOT]
    sem_s = sems[ISLOT + NSLOT:]

    pltpu.sync_copy(w3_hbm, w3v)

    r0 = s * ROWS_PER_SUB
    pltpu.sync_copy(zero_hbm.at[pl.ds(r0, ROWS_PER_SUB)],
                    agg_sh.at[pl.ds(r0, ROWS_PER_SUB)])

    @pl.when(s == NS - 1)
    def _zero_tail():
        pltpu.sync_copy(zero_hbm.at[pl.ds(NS * ROWS_PER_SUB, ROWS_TAIL)],
                        agg_sh.at[pl.ds(NS * ROWS_PER_SUB, ROWS_TAIL)])

    plsc.subcore_barrier()

    def idx_cps(g, q):
        base = wid * E_PER_W + g * CH
        return (
            pltpu.make_async_copy(src_hbm.at[pl.ds(base, CH)], si.at[q], sem_i[q]),
            pltpu.make_async_copy(dst_hbm.at[pl.ds(base, CH)], di.at[q], sem_i[q]),
        )

    def gather_cps(g, k, q):
        base = wid * E_PER_W + g * CH
        return (
            pltpu.make_async_copy(a_hbm.at[si.at[q]], av.at[k], sem_g[k]),
            pltpu.make_async_copy(b_hbm.at[di.at[q]], bv.at[k], sem_g[k]),
            pltpu.make_async_copy(ea_hbm.at[pl.ds(base * 4, CH * 4 + 16)],
                                  evs[k], sem_g[k]),
        )

    def scatter_start(k, q):
        pltpu.async_copy(av.at[k], agg_sh.at[di.at[q]], sem_s[k], add=True)

    def scatter_wait(k, q):
        pltpu.make_async_copy(av.at[k], agg_sh.at[di.at[q]], sem_s[k]).wait()

    w3 = [[w3v[kk, pl.ds(j * 16, 16)] for j in range(H // 16)]
          for kk in range(4)]
    lane0 = [jnp.full((16, 1), kk, jnp.int32) for kk in range(4)]
    _dnums = lax.GatherDimensionNumbers(
        offset_dims=(), collapsed_slice_dims=(0,), start_index_map=(0,))

    def _splat(vec, idx):
        return lax.gather(vec, idx, _dnums, (1,),
                          mode=lax.GatherScatterMode.PROMISE_IN_BOUNDS)

    for g in range(3):
        for cp in idx_cps(g, g):
            cp.start()
    for g in range(2):
        for cp in idx_cps(g, g):
            cp.wait()
        for cp in gather_cps(g, g, g):
            cp.start()

    def six(i, carry):
        for u in range(ISLOT):
            g = i * ISLOT + u
            k = u % NSLOT
            q = u
            kp = (u + 2) % NSLOT
            qn = (u + 2) % ISLOT
            qf = (u + 3) % ISLOT

            @pl.when(g < NCHUNK)
            def _chunk():
                for cp in gather_cps(g, k, q):
                    cp.wait()

                def edge(e, carry2):
                    v4 = evs[k][pl.ds(e * 4, 16)]
                    spl = [_splat(v4, lane0[kk]) for kk in range(4)]
                    for j in range(H // 16):
                        sl = pl.ds(j * 16, 16)
                        v = av[k, e, sl] + bv[k, e, sl]
                        for kk in range(4):
                            v = v + spl[kk] * w3[kk][j]
                        av[k, e, sl] = jnp.maximum(v, 0.0)
                    return carry2

                lax.fori_loop(0, CH, edge, 0)
                scatter_start(k, q)

                @pl.when(g >= 1)
                def _drain_prev():
                    scatter_wait(kp, (u + ISLOT - 1) % ISLOT)

                @pl.when(g + 3 < NCHUNK)
                def _idx_ahead():
                    for cp in idx_cps(g + 3, qf):
                        cp.start()

                @pl.when(g + 2 < NCHUNK)
                def _refill():
                    for cp in idx_cps(g + 2, qn):
                        cp.wait()
                    for cp in gather_cps(g + 2, kp, qn):
                        cp.start()

        return carry

    lax.fori_loop(0, (NCHUNK + ISLOT - 1) // ISLOT, six, 0)
    scatter_wait((NCHUNK - 1) % NSLOT, (NCHUNK - 1) % ISLOT)
    plsc.subcore_barrier()
    pltpu.sync_copy(agg_sh.at[pl.ds(r0, ROWS_PER_SUB)],
                    out_hbm.at[pl.ds(c * N + r0, ROWS_PER_SUB)])

    @pl.when(s == NS - 1)
    def _out_tail():
        pltpu.sync_copy(agg_sh.at[pl.ds(NS * ROWS_PER_SUB, ROWS_TAIL)],
                        out_hbm.at[pl.ds(c * N + NS * ROWS_PER_SUB, ROWS_TAIL)])


_edge_pass = functools.partial(
    pl.kernel,
    mesh=plsc.VectorSubcoreMesh(core_axis_name="c", subcore_axis_name="s"),
    out_type=jax.ShapeDtypeStruct((2 * N, H), jnp.float32),
    scratch_types=[
        pltpu.VMEM((ISLOT, CH), jnp.int32),
        pltpu.VMEM((ISLOT, CH), jnp.int32),
        pltpu.VMEM((NSLOT, CH, H), jnp.float32),
        pltpu.VMEM((NSLOT, CH, H), jnp.float32),
        pltpu.VMEM((CH * 4 + 16,), jnp.float32),
        pltpu.VMEM((CH * 4 + 16,), jnp.float32),
        pltpu.VMEM((CH * 4 + 16,), jnp.float32),
        pltpu.VMEM((8, H), jnp.float32),
        pltpu.VMEM_SHARED((N, H), jnp.float32),
    ] + [pltpu.SemaphoreType.DMA] * (ISLOT + 2 * NSLOT),
)(_edge_body)


_RB = 400
_GRID = N // _RB


def _enc_body(x, W_enc, b_enc, We1, We2, b_edge, h_o, a_o, b_o):
    h = jnp.dot(x[...], W_enc[...], preferred_element_type=jnp.float32) + b_enc[...]
    h_o[...] = h
    a_o[...] = jnp.dot(h, We1[...], preferred_element_type=jnp.float32) + b_edge[...]
    b_o[...] = jnp.dot(h, We2[...], preferred_element_type=jnp.float32)


def _enc_ab(x, W_enc, b_enc, We1, We2, b_edge):
    w = pl.BlockSpec((H, H), lambda i: (0, 0))
    b = pl.BlockSpec((1, H), lambda i: (0, 0))
    return pl.pallas_call(
        _enc_body,
        grid=(_GRID,),
        in_specs=[pl.BlockSpec((_RB, H), lambda i: (i, 0)), w, b, w, w, b],
        out_specs=[pl.BlockSpec((_RB, H), lambda i: (i, 0))] * 3,
        out_shape=[jax.ShapeDtypeStruct((N, H), jnp.float32)] * 3,
    )(x, W_enc, b_enc, We1, We2, b_edge)


def _node_body(h, p0, p1, Wn1, Wn2, b_node, We1, We2, b_edge, h_o, a_o, b_o):
    agg = p0[...] + p1[...]
    t = (jnp.dot(h[...], Wn1[...], preferred_element_type=jnp.float32)
         + jnp.dot(agg, Wn2[...], preferred_element_type=jnp.float32)
         + b_node[...])
    t = jnp.maximum(t, 0.0)
    h_o[...] = t
    a_o[...] = jnp.dot(t, We1[...], preferred_element_type=jnp.float32) + b_edge[...]
    b_o[...] = jnp.dot(t, We2[...], preferred_element_type=jnp.float32)


def _node_update(h, p0, p1, Wn1, Wn2, b_node, We1, We2, b_edge):
    r = pl.BlockSpec((_RB, H), lambda i: (i, 0))
    w = pl.BlockSpec((H, H), lambda i: (0, 0))
    b = pl.BlockSpec((1, H), lambda i: (0, 0))
    return pl.pallas_call(
        _node_body,
        grid=(_GRID,),
        in_specs=[r, r, r, w, w, b, w, w, b],
        out_specs=[r] * 3,
        out_shape=[jax.ShapeDtypeStruct((N, H), jnp.float32)] * 3,
    )(h, p0, p1, Wn1, Wn2, b_node, We1, We2, b_edge)


def _final_body(h, p0, p1, Wn1, Wn2, b_node, Wd1, bd1, Wd2, bd2, out_o):
    agg = p0[...] + p1[...]
    t = (jnp.dot(h[...], Wn1[...], preferred_element_type=jnp.float32)
         + jnp.dot(agg, Wn2[...], preferred_element_type=jnp.float32)
         + b_node[...])
    t = jnp.maximum(t, 0.0)
    z = jnp.maximum(
        jnp.dot(t, Wd1[...], preferred_element_type=jnp.float32) + bd1[...], 0.0)
    out_o[...] = jnp.dot(z, Wd2[...], preferred_element_type=jnp.float32) + bd2[...]


def _final_dec(h, p0, p1, Wn1, Wn2, b_node, Wd1, bd1, Wd2, bd2, DT, DO):
    r = pl.BlockSpec((_RB, H), lambda i: (i, 0))
    w = pl.BlockSpec((H, H), lambda i: (0, 0))
    return pl.pallas_call(
        _final_body,
        grid=(_GRID,),
        in_specs=[r, r, r, w, w, pl.BlockSpec((1, H), lambda i: (0, 0)),
                  pl.BlockSpec((H, DT), lambda i: (0, 0)),
                  pl.BlockSpec((1, DT), lambda i: (0, 0)),
                  pl.BlockSpec((DT, DO), lambda i: (0, 0)),
                  pl.BlockSpec((1, DO), lambda i: (0, 0))],
        out_specs=pl.BlockSpec((_RB, DO), lambda i: (i, 0)),
        out_shape=jax.ShapeDtypeStruct((N, DO), jnp.float32),
    )(h, p0, p1, Wn1, Wn2, b_node, Wd1, bd1, Wd2, bd2)


def kernel(x, edge_index, edge_attr, W_enc, b_enc, W_edge, b_edge,
           W_node, b_node, dec):
    We1 = W_edge[:H]
    We2 = W_edge[H:2 * H]
    We3 = jnp.concatenate([W_edge[2 * H:], jnp.zeros((4, H), jnp.float32)])
    Wn1 = W_node[:H]
    Wn2 = W_node[H:]
    src = edge_index[0]
    dst = edge_index[1]
    b_enc2 = b_enc.reshape(1, H)
    b_edge2 = b_edge.reshape(1, H)
    b_node2 = b_node.reshape(1, H)

    names = sorted(dec.keys())
    dims = [dec[n][2].shape[1] for n in names]
    DT = 64 * len(names)
    DO = sum(dims)
    Wd1 = jnp.concatenate([dec[n][0] for n in names], axis=1)
    bd1 = jnp.concatenate([dec[n][1] for n in names]).reshape(1, DT)
    Wd2 = jnp.zeros((DT, DO), jnp.float32)
    off = 0
    for i, n in enumerate(names):
        Wd2 = Wd2.at[64 * i:64 * (i + 1), off:off + dims[i]].set(dec[n][2])
        off += dims[i]
    bd2 = jnp.concatenate([dec[n][3] for n in names]).reshape(1, DO)

    zero = jnp.zeros((N, H), jnp.float32)
    ea_flat = jnp.concatenate([edge_attr.reshape(E * 4),
                               jnp.zeros((16,), jnp.float32)])

    h, A, B = _enc_ab(x, W_enc, b_enc2, We1, We2, b_edge2)

    for layer in range(3):
        parts = _edge_pass(A, B, ea_flat, We3, src, dst, zero)
        p0, p1 = parts[:N], parts[N:]
        if layer < 2:
            h, A, B = _node_update(h, p0, p1, Wn1, Wn2, b_node2, We1, We2,
                                   b_edge2)
        else:
            out = _final_dec(h, p0, p1, Wn1, Wn2, b_node2, Wd1, bd1, Wd2, bd2,
                             DT, DO)
    return out
